# baseline (speedup 1.0000x reference)
import functools

import jax
import jax.numpy as jnp
from jax import lax
from jax.experimental import pallas as pl
from jax.experimental.pallas import tpu as pltpu
from jax.experimental.pallas import tpu_sc as plsc

N = 10000
E = 320000
IN = 128
H = 64
OUT = 64
C = 40

NC = 2
NS = 16
NW = NC * NS
BATCH = 128
EPT = 10112
STEPS = EPT // BATCH
E_PAD = EPT * NW
N_PAD = 10112
STRIPE = N_PAD // NS
ROWS_BLK = 1000
NBUF = 6
GAHEAD = 3
SLAG = 3


def _tc_in_proj(x, Wl1, Wr1):
    def body(x_ref, wl_ref, wr_ref, o1_ref, o2_ref):
        xb = x_ref[...]
        dn = (((1,), (1,)), ((), ()))
        o1_ref[...] = lax.dot_general(xb, wl_ref[...], dn,
                                      preferred_element_type=jnp.float32)
        o2_ref[...] = lax.dot_general(xb, wr_ref[...], dn,
                                      preferred_element_type=jnp.float32)

    grid = (N // ROWS_BLK,)
    return pl.pallas_call(
        body,
        grid=grid,
        in_specs=[
            pl.BlockSpec((ROWS_BLK, IN), lambda i: (i, 0)),
            pl.BlockSpec((H, IN), lambda i: (0, 0)),
            pl.BlockSpec((H, IN), lambda i: (0, 0)),
        ],
        out_specs=[
            pl.BlockSpec((ROWS_BLK, H), lambda i: (i, 0)),
            pl.BlockSpec((ROWS_BLK, H), lambda i: (i, 0)),
        ],
        out_shape=[
            jax.ShapeDtypeStruct((N, H), jnp.float32),
            jax.ShapeDtypeStruct((N, H), jnp.float32),
        ],
    )(x, Wl1, Wr1)


def _sc_segment_sum(table, src_r, dst_r, z_acc, ones_b, z_cnt):
    D = table.shape[1]
    count = ones_b is not None
    mesh = plsc.VectorSubcoreMesh(core_axis_name="c", subcore_axis_name="s")

    out_type = [jax.ShapeDtypeStruct((NC, N_PAD, D), jnp.float32)]
    scratch = [
        pltpu.VMEM((STEPS, BATCH), jnp.int32),
        pltpu.VMEM((STEPS, BATCH), jnp.int32),
        pltpu.VMEM((NBUF, BATCH, D), jnp.float32),
        pltpu.VMEM_SHARED((N_PAD, D), jnp.float32),
        pltpu.SemaphoreType.DMA,
        pltpu.SemaphoreType.DMA,
    ]
    if count:
        out_type.append(jax.ShapeDtypeStruct((NC, N_PAD, 8), jnp.float32))
        scratch.append(pltpu.VMEM((BATCH, 8), jnp.float32))
        scratch.append(pltpu.VMEM_SHARED((N_PAD, 8), jnp.float32))
        scratch.append(pltpu.SemaphoreType.DMA)

    @functools.partial(
        pl.kernel,
        mesh=mesh,
        out_type=out_type,
        scratch_types=scratch,
        compiler_params=pltpu.CompilerParams(use_tc_tiling_on_sc=False),
    )
    def k(*refs):
        if count:
            (table_h, src_h, dst_h, zacc_h, ones_h, zcnt_h,
             out_h, cnt_h, src_v, dst_v, rows_v, acc_s, gsem, ssem,
             ones_v, cacc_s, csem) = refs
        else:
            (table_h, src_h, dst_h, zacc_h,
             out_h, src_v, dst_v, rows_v, acc_s, gsem, ssem) = refs
        c = lax.axis_index("c")
        s = lax.axis_index("s")
        wid = s * NC + c
        r0 = s * STRIPE

        pltpu.sync_copy(src_h.at[wid], src_v)
        pltpu.sync_copy(dst_h.at[wid], dst_v)
        pltpu.sync_copy(zacc_h.at[pl.ds(r0, STRIPE)], acc_s.at[pl.ds(r0, STRIPE)])
        if count:
            pltpu.sync_copy(ones_h, ones_v)
            pltpu.sync_copy(zcnt_h.at[pl.ds(r0, STRIPE)],
                            cacc_s.at[pl.ds(r0, STRIPE)])
        plsc.subcore_barrier()

        for jj in range(GAHEAD):
            pltpu.async_copy(table_h.at[src_v.at[jj]], rows_v.at[jj], gsem)

        def step(j, carry):
            p = lax.rem(j, NBUF)

            @pl.when(j >= SLAG)
            def _():
                jo = j - SLAG
                po = lax.rem(jo, NBUF)
                pltpu.make_async_copy(rows_v.at[po],
                                      acc_s.at[dst_v.at[jo]], ssem).wait()
                if count:
                    pltpu.make_async_copy(ones_v,
                                          cacc_s.at[dst_v.at[jo]],
                                          csem).wait()

            @pl.when(j + GAHEAD < STEPS)
            def _():
                jn = j + GAHEAD
                pltpu.async_copy(table_h.at[src_v.at[jn]],
                                 rows_v.at[lax.rem(jn, NBUF)], gsem)

            pltpu.make_async_copy(table_h.at[src_v.at[j]],
                                  rows_v.at[p], gsem).wait()
            pltpu.async_copy(rows_v.at[p], acc_s.at[dst_v.at[j]], ssem,
                             add=True)
            if count:
                pltpu.async_copy(ones_v, cacc_s.at[dst_v.at[j]], csem,
                                 add=True)
            return carry

        lax.fori_loop(0, STEPS, step, 0)
        for jj in range(STEPS - SLAG, STEPS):
            pltpu.make_async_copy(rows_v.at[jj % NBUF],
                                  acc_s.at[dst_v.at[jj]], ssem).wait()
            if count:
                pltpu.make_async_copy(ones_v, cacc_s.at[dst_v.at[jj]],
                                      csem).wait()
        plsc.subcore_barrier()

        pltpu.sync_copy(acc_s.at[pl.ds(r0, STRIPE)],
                        out_h.at[c, pl.ds(r0, STRIPE)])
        if count:
            pltpu.sync_copy(cacc_s.at[pl.ds(r0, STRIPE)],
                            cnt_h.at[c, pl.ds(r0, STRIPE)])

    if count:
        res = k(table, src_r, dst_r, z_acc, ones_b, z_cnt)
    else:
        res = k(table, src_r, dst_r, z_acc)
    if isinstance(res, (list, tuple)):
        return tuple(res)
    return (res,)


def _tc_mid(f1_part, cnt_part, xr1, bl1_2d, Wl2, Wr2, Wc):
    def body(f_ref, c_ref, xr_ref, b_ref,
             wl2_ref, wr2_ref, wc_ref, oa_ref, ob_ref):
        aggsum = f_ref[0] + f_ref[1]
        cnt = c_ref[0, :, 0:1] + c_ref[1, :, 0:1]
        inv = 1.0 / jnp.maximum(cnt, 1.0)
        h = jnp.maximum(aggsum * inv + b_ref[0:1, :] + xr_ref[...], 0.0)
        MA = jnp.dot(wc_ref[...], wl2_ref[...],
                     preferred_element_type=jnp.float32)
        MB = jnp.dot(wc_ref[...], wr2_ref[...],
                     preferred_element_type=jnp.float32)
        dn = (((1,), (1,)), ((), ()))
        hA = lax.dot_general(h, MA, dn, preferred_element_type=jnp.float32)
        hB = lax.dot_general(h, MB, dn, preferred_element_type=jnp.float32)
        pad = jnp.zeros((hA.shape[0], 8), jnp.float32)
        oa_ref[...] = jnp.concatenate([hA, pad], axis=1)
        ob_ref[...] = jnp.concatenate([hB, pad], axis=1)

    grid = (N // ROWS_BLK,)
    blk = lambda d: pl.BlockSpec((ROWS_BLK, d), lambda i: (i, 0))
    pblk = lambda d: pl.BlockSpec((2, ROWS_BLK, d), lambda i: (0, i, 0))
    full = lambda a, b: pl.BlockSpec((a, b), lambda i: (0, 0))
    return pl.pallas_call(
        body,
        grid=grid,
        in_specs=[pblk(H), pblk(8), blk(H), full(8, H),
                  full(OUT, H), full(OUT, H), full(C, OUT)],
        out_specs=[blk(48), blk(48)],
        out_shape=[
            jax.ShapeDtypeStruct((N, 48), jnp.float32),
            jax.ShapeDtypeStruct((N, 48), jnp.float32),
        ],
    )(f1_part, cnt_part, xr1, bl1_2d, Wl2, Wr2, Wc)


def _tc_out(f2_part, cnt_part, hB, bl2_2d, bc_2d, Wc):
    def body(f_ref, c_ref, hb_ref, bl_ref, bc_ref, wc_ref,
             o_ref):
        aggsum = f_ref[0] + f_ref[1]
        cnt = c_ref[0, :, 0:1] + c_ref[1, :, 0:1]
        inv = 1.0 / jnp.maximum(cnt, 1.0)
        dn = (((1,), (1,)), ((), ()))
        c2 = lax.dot_general(bl_ref[0:1, :], wc_ref[...], dn,
                             preferred_element_type=jnp.float32)
        o_ref[...] = (aggsum[:, :C] * inv + hb_ref[:, :C]
                      + c2 + bc_ref[0:1, :])

    grid = (N // ROWS_BLK,)
    blk = lambda d: pl.BlockSpec((ROWS_BLK, d), lambda i: (i, 0))
    pblk = lambda d: pl.BlockSpec((2, ROWS_BLK, d), lambda i: (0, i, 0))
    full = lambda a, b: pl.BlockSpec((a, b), lambda i: (0, 0))
    return pl.pallas_call(
        body,
        grid=grid,
        in_specs=[pblk(48), pblk(8), blk(48), full(8, OUT),
                  full(8, C), full(C, OUT)],
        out_specs=blk(C),
        out_shape=jax.ShapeDtypeStruct((N, C), jnp.float32),
    )(f2_part, cnt_part, hB, bl2_2d, bc_2d, Wc)


def kernel(x, edge_index, Wl1, bl1, Wr1, Wl2, bl2, Wr2, Wc, bc):
    src = edge_index[0]
    dst = edge_index[1]
    pad = E_PAD - E
    src_r = jnp.concatenate([src, jnp.zeros((pad,), jnp.int32)]
                            ).reshape(NW, STEPS, BATCH)
    dst_r = jnp.concatenate([dst, jnp.full((pad,), N, jnp.int32)]
                            ).reshape(NW, STEPS, BATCH)
    z64 = jnp.zeros((N_PAD, H), jnp.float32)
    z48 = jnp.zeros((N_PAD, 48), jnp.float32)
    z8 = jnp.zeros((N_PAD, 8), jnp.float32)
    ones_b = jnp.ones((BATCH, 8), jnp.float32)
    bl1_2d = jnp.broadcast_to(bl1[None, :], (8, H))
    bl2_2d = jnp.broadcast_to(bl2[None, :], (8, OUT))
    bc_2d = jnp.broadcast_to(bc[None, :], (8, C))

    xw1, xr1 = _tc_in_proj(x, Wl1, Wr1)

    f1_part, cnt_part = _sc_segment_sum(xw1, src_r, dst_r, z64, ones_b, z8)

    hA, hB = _tc_mid(f1_part, cnt_part, xr1, bl1_2d, Wl2, Wr2, Wc)

    (f2_part,) = _sc_segment_sum(hA, src_r, dst_r, z48, None, None)

    return _tc_out(f2_part, cnt_part, hB, bl2_2d, bc_2d, Wc)


# speedup vs baseline: 1.4590x; 1.4590x over previous
import functools

import jax
import jax.numpy as jnp
from jax import lax
from jax.experimental import pallas as pl
from jax.experimental.pallas import tpu as pltpu
from jax.experimental.pallas import tpu_sc as plsc

N = 10000
E = 320000
IN = 128
H = 64
OUT = 64
C = 40

NC = 2
NS = 16
BATCH = 128
STEPS = 157
EPT = STEPS * BATCH
E_PAD = EPT * NS
N_PAD = 10016
STRIPE = N_PAD // NS
TSTRIPE = N // NS
ROWS_BLK = 1000
NBUF = 6
GAHEAD = 3
SLAG = 3
DH1 = H // 2
CH = C // 2
DH2 = 24


def _tc_in_proj(x, Wl1, Wr1):
    def body(x_ref, wl_ref, wr_ref, o1_ref, o2_ref):
        xb = x_ref[...]
        dn = (((1,), (1,)), ((), ()))
        r1 = lax.dot_general(xb, wl_ref[...], dn,
                             preferred_element_type=jnp.float32)
        o1_ref[0] = r1[:, :DH1]
        o1_ref[1] = r1[:, DH1:]
        o2_ref[...] = lax.dot_general(xb, wr_ref[...], dn,
                                      preferred_element_type=jnp.float32)

    grid = (N // ROWS_BLK,)
    return pl.pallas_call(
        body,
        grid=grid,
        in_specs=[
            pl.BlockSpec((ROWS_BLK, IN), lambda i: (i, 0)),
            pl.BlockSpec((H, IN), lambda i: (0, 0)),
            pl.BlockSpec((H, IN), lambda i: (0, 0)),
        ],
        out_specs=[
            pl.BlockSpec((2, ROWS_BLK, DH1), lambda i: (0, i, 0)),
            pl.BlockSpec((ROWS_BLK, H), lambda i: (i, 0)),
        ],
        out_shape=[
            jax.ShapeDtypeStruct((2, N, DH1), jnp.float32),
            jax.ShapeDtypeStruct((N, H), jnp.float32),
        ],
    )(x, Wl1, Wr1)


def _sc_segment_sum(table, src_r, dst_r, z_acc, ones_b, z_cnt):
    D = table.shape[2]
    count = ones_b is not None
    mesh = plsc.VectorSubcoreMesh(core_axis_name="c", subcore_axis_name="s")

    out_type = [jax.ShapeDtypeStruct((NC, N_PAD, D), jnp.float32)]
    scratch = [
        pltpu.VMEM((STEPS, BATCH), jnp.int32),
        pltpu.VMEM((STEPS, BATCH), jnp.int32),
        pltpu.VMEM((NBUF, BATCH, D), jnp.float32),
        pltpu.VMEM_SHARED((N, D), jnp.float32),
        pltpu.VMEM_SHARED((N_PAD, D), jnp.float32),
        pltpu.SemaphoreType.DMA,
        pltpu.SemaphoreType.DMA,
    ]
    if count:
        out_type.append(jax.ShapeDtypeStruct((N_PAD, 8), jnp.float32))
        scratch.append(pltpu.VMEM((BATCH, 8), jnp.float32))
        scratch.append(pltpu.VMEM_SHARED((N_PAD, 8), jnp.float32))
        scratch.append(pltpu.SemaphoreType.DMA)

    @functools.partial(
        pl.kernel,
        mesh=mesh,
        out_type=out_type,
        scratch_types=scratch,
        compiler_params=pltpu.CompilerParams(use_tc_tiling_on_sc=False),
    )
    def k(*refs):
        if count:
            (table_h, src_h, dst_h, zacc_h, ones_h, zcnt_h,
             out_h, cnt_h, src_v, dst_v, rows_v, table_s, acc_s, gsem, ssem,
             ones_v, cacc_s, csem) = refs
        else:
            (table_h, src_h, dst_h, zacc_h,
             out_h, src_v, dst_v, rows_v, table_s, acc_s, gsem, ssem) = refs
        c = lax.axis_index("c")
        s = lax.axis_index("s")
        r0 = s * STRIPE
        t0 = s * TSTRIPE

        pltpu.sync_copy(src_h.at[s], src_v)
        pltpu.sync_copy(dst_h.at[s], dst_v)
        pltpu.sync_copy(table_h.at[c, pl.ds(t0, TSTRIPE)],
                        table_s.at[pl.ds(t0, TSTRIPE)])
        pltpu.sync_copy(zacc_h.at[pl.ds(r0, STRIPE)], acc_s.at[pl.ds(r0, STRIPE)])
        if count:
            @pl.when(c == 1)
            def _():
                pltpu.sync_copy(ones_h, ones_v)
                pltpu.sync_copy(zcnt_h.at[pl.ds(r0, STRIPE)],
                                cacc_s.at[pl.ds(r0, STRIPE)])
        plsc.subcore_barrier()

        for jj in range(GAHEAD):
            pltpu.async_copy(table_s.at[src_v.at[jj]], rows_v.at[jj], gsem)

        def step(j, carry):
            p = lax.rem(j, NBUF)

            @pl.when(j >= SLAG)
            def _():
                jo = j - SLAG
                po = lax.rem(jo, NBUF)
                pltpu.make_async_copy(rows_v.at[po],
                                      acc_s.at[dst_v.at[jo]], ssem).wait()
                if count:
                    @pl.when(c == 1)
                    def _():
                        pltpu.make_async_copy(ones_v,
                                              cacc_s.at[dst_v.at[jo]],
                                              csem).wait()

            @pl.when(j + GAHEAD < STEPS)
            def _():
                jn = j + GAHEAD
                pltpu.async_copy(table_s.at[src_v.at[jn]],
                                 rows_v.at[lax.rem(jn, NBUF)], gsem)

            pltpu.make_async_copy(table_s.at[src_v.at[j]],
                                  rows_v.at[p], gsem).wait()
            pltpu.async_copy(rows_v.at[p], acc_s.at[dst_v.at[j]], ssem,
                             add=True)
            if count:
                @pl.when(c == 1)
                def _():
                    pltpu.async_copy(ones_v, cacc_s.at[dst_v.at[j]], csem,
                                     add=True)
            return carry

        lax.fori_loop(0, STEPS, step, 0)
        for jj in range(STEPS - SLAG, STEPS):
            pltpu.make_async_copy(rows_v.at[jj % NBUF],
                                  acc_s.at[dst_v.at[jj]], ssem).wait()
            if count:
                @pl.when(c == 1)
                def _():
                    pltpu.make_async_copy(ones_v, cacc_s.at[dst_v.at[jj]],
                                          csem).wait()
        plsc.subcore_barrier()

        pltpu.sync_copy(acc_s.at[pl.ds(r0, STRIPE)],
                        out_h.at[c, pl.ds(r0, STRIPE)])
        if count:
            @pl.when(c == 1)
            def _():
                pltpu.sync_copy(cacc_s.at[pl.ds(r0, STRIPE)],
                                cnt_h.at[pl.ds(r0, STRIPE)])

    if count:
        res = k(table, src_r, dst_r, z_acc, ones_b, z_cnt)
    else:
        res = k(table, src_r, dst_r, z_acc)
    if isinstance(res, (list, tuple)):
        return tuple(res)
    return (res,)


def _tc_mid(f1, cnt8, xr1, bl1_2d, Wl2, Wr2, Wc):
    def body(f_ref, c_ref, xr_ref, b_ref,
             wl2_ref, wr2_ref, wc_ref, oa_ref, ob_ref):
        aggsum = jnp.concatenate([f_ref[0], f_ref[1]], axis=1)
        cnt = c_ref[:, 0:1]
        inv = 1.0 / jnp.maximum(cnt, 1.0)
        h = jnp.maximum(aggsum * inv + b_ref[0:1, :] + xr_ref[...], 0.0)
        MA = jnp.dot(wc_ref[...], wl2_ref[...],
                     preferred_element_type=jnp.float32)
        MB = jnp.dot(wc_ref[...], wr2_ref[...],
                     preferred_element_type=jnp.float32)
        dn = (((1,), (1,)), ((), ()))
        hA = lax.dot_general(h, MA, dn, preferred_element_type=jnp.float32)
        hB = lax.dot_general(h, MB, dn, preferred_element_type=jnp.float32)
        pad4 = jnp.zeros((hA.shape[0], DH2 - CH), jnp.float32)
        oa_ref[0] = jnp.concatenate([hA[:, :CH], pad4], axis=1)
        oa_ref[1] = jnp.concatenate([hA[:, CH:], pad4], axis=1)
        pad8 = jnp.zeros((hB.shape[0], 8), jnp.float32)
        ob_ref[...] = jnp.concatenate([hB, pad8], axis=1)

    grid = (N // ROWS_BLK,)
    blk = lambda d: pl.BlockSpec((ROWS_BLK, d), lambda i: (i, 0))
    full = lambda a, b: pl.BlockSpec((a, b), lambda i: (0, 0))
    return pl.pallas_call(
        body,
        grid=grid,
        in_specs=[pl.BlockSpec((2, ROWS_BLK, DH1), lambda i: (0, i, 0)),
                  blk(8), blk(H), full(8, H),
                  full(OUT, H), full(OUT, H), full(C, OUT)],
        out_specs=[pl.BlockSpec((2, ROWS_BLK, DH2), lambda i: (0, i, 0)),
                   blk(48)],
        out_shape=[
            jax.ShapeDtypeStruct((2, N, DH2), jnp.float32),
            jax.ShapeDtypeStruct((N, 48), jnp.float32),
        ],
    )(f1, cnt8, xr1, bl1_2d, Wl2, Wr2, Wc)


def _tc_out(f2, cnt8, hB, bl2_2d, bc_2d, Wc):
    def body(f_ref, c_ref, hb_ref, bl_ref, bc_ref, wc_ref, o_ref):
        aggsum = jnp.concatenate([f_ref[0][:, :CH], f_ref[1][:, :CH]], axis=1)
        cnt = c_ref[:, 0:1]
        inv = 1.0 / jnp.maximum(cnt, 1.0)
        dn = (((1,), (1,)), ((), ()))
        c2 = lax.dot_general(bl_ref[0:1, :], wc_ref[...], dn,
                             preferred_element_type=jnp.float32)
        o_ref[...] = (aggsum * inv + hb_ref[:, :C]
                      + c2 + bc_ref[0:1, :])

    grid = (N // ROWS_BLK,)
    blk = lambda d: pl.BlockSpec((ROWS_BLK, d), lambda i: (i, 0))
    full = lambda a, b: pl.BlockSpec((a, b), lambda i: (0, 0))
    return pl.pallas_call(
        body,
        grid=grid,
        in_specs=[pl.BlockSpec((2, ROWS_BLK, DH2), lambda i: (0, i, 0)),
                  blk(8), blk(48), full(8, OUT),
                  full(8, C), full(C, OUT)],
        out_specs=blk(C),
        out_shape=jax.ShapeDtypeStruct((N, C), jnp.float32),
    )(f2, cnt8, hB, bl2_2d, bc_2d, Wc)


def kernel(x, edge_index, Wl1, bl1, Wr1, Wl2, bl2, Wr2, Wc, bc):
    src = edge_index[0]
    dst = edge_index[1]
    pad = E_PAD - E
    src_r = jnp.concatenate([src, jnp.zeros((pad,), jnp.int32)]
                            ).reshape(NS, STEPS, BATCH)
    dst_r = jnp.concatenate([dst, jnp.full((pad,), N, jnp.int32)]
                            ).reshape(NS, STEPS, BATCH)
    z32 = jnp.zeros((N_PAD, DH1), jnp.float32)
    z24 = jnp.zeros((N_PAD, DH2), jnp.float32)
    z8 = jnp.zeros((N_PAD, 8), jnp.float32)
    ones_b = jnp.ones((BATCH, 8), jnp.float32)
    bl1_2d = jnp.broadcast_to(bl1[None, :], (8, H))
    bl2_2d = jnp.broadcast_to(bl2[None, :], (8, OUT))
    bc_2d = jnp.broadcast_to(bc[None, :], (8, C))

    xw1, xr1 = _tc_in_proj(x, Wl1, Wr1)

    f1, cnt8 = _sc_segment_sum(xw1, src_r, dst_r, z32, ones_b, z8)

    hA, hB = _tc_mid(f1, cnt8, xr1, bl1_2d, Wl2, Wr2, Wc)

    (f2,) = _sc_segment_sum(hA, src_r, dst_r, z24, None, None)

    return _tc_out(f2, cnt8, hB, bl2_2d, bc_2d, Wc)


# speedup vs baseline: 1.6008x; 1.0972x over previous
import functools

import jax
import jax.numpy as jnp
from jax import lax
from jax.experimental import pallas as pl
from jax.experimental.pallas import tpu as pltpu
from jax.experimental.pallas import tpu_sc as plsc

N = 10000
E = 320000
IN = 128
H = 64
OUT = 64
C = 40

NC = 2
NS = 16
BATCH = 128
STEPS = 157
EPT = STEPS * BATCH
E_PAD = EPT * NS
N_PAD = 10016
STRIPE = N_PAD // NS
TSTRIPE = N // NS
ROWS_BLK = 2000
NBUF = 6
GAHEAD = 3
SLAG = 3
DH1 = H // 2
CH = C // 2
DH2 = 24


def _tc_in_proj(x, Wl1, Wr1):
    def body(x_ref, wl_ref, wr_ref, o_ref):
        xb = x_ref[...]
        dn = (((1,), (1,)), ((), ()))
        r1 = lax.dot_general(xb, wl_ref[...], dn,
                             preferred_element_type=jnp.float32)
        r2 = lax.dot_general(xb, wr_ref[...], dn,
                             preferred_element_type=jnp.float32)
        o_ref[...] = jnp.concatenate([r1, r2], axis=1)

    grid = (N // ROWS_BLK,)
    return pl.pallas_call(
        body,
        grid=grid,
        in_specs=[
            pl.BlockSpec((ROWS_BLK, IN), lambda i: (i, 0)),
            pl.BlockSpec((H, IN), lambda i: (0, 0)),
            pl.BlockSpec((H, IN), lambda i: (0, 0)),
        ],
        out_specs=pl.BlockSpec((ROWS_BLK, 128), lambda i: (i, 0)),
        out_shape=jax.ShapeDtypeStruct((N, 128), jnp.float32),
    )(x, Wl1, Wr1)


def _sc_segment_mean(table, col_off, D, src_r, dst_r, z_acc, ones_b, z_cnt,
                     cnt_in):
    count = ones_b is not None
    mesh = plsc.VectorSubcoreMesh(core_axis_name="c", subcore_axis_name="s")

    out_type = [jax.ShapeDtypeStruct((NC, N_PAD, D), jnp.float32)]
    scratch = [
        pltpu.VMEM((STEPS, BATCH), jnp.int32),
        pltpu.VMEM((STEPS, BATCH), jnp.int32),
        pltpu.VMEM((NBUF, BATCH, D), jnp.float32),
        pltpu.VMEM_SHARED((N, D), jnp.float32),
        pltpu.VMEM_SHARED((N_PAD, D), jnp.float32),
        pltpu.SemaphoreType.DMA,
        pltpu.SemaphoreType.DMA,
    ]
    if count:
        out_type.append(jax.ShapeDtypeStruct((N_PAD, 8), jnp.float32))
        scratch.append(pltpu.VMEM((BATCH, 8), jnp.float32))
        scratch.append(pltpu.VMEM_SHARED((N_PAD, 8), jnp.float32))
        scratch.append(pltpu.SemaphoreType.DMA)

    @functools.partial(
        pl.kernel,
        mesh=mesh,
        out_type=out_type,
        scratch_types=scratch,
        compiler_params=pltpu.CompilerParams(use_tc_tiling_on_sc=False),
    )
    def k(*refs):
        if count:
            (table_h, src_h, dst_h, zacc_h, ones_h, zcnt_h,
             out_h, cnt_h, src_v, dst_v, rows_v,
             table_s, acc_s, gsem, ssem, ones_v, cacc_s, csem) = refs
        else:
            (table_h, src_h, dst_h, zacc_h,
             out_h, src_v, dst_v, rows_v,
             table_s, acc_s, gsem, ssem) = refs
        c = lax.axis_index("c")
        s = lax.axis_index("s")
        r0 = s * STRIPE
        t0 = s * TSTRIPE
        co = col_off * c

        pltpu.sync_copy(src_h.at[s], src_v)
        pltpu.sync_copy(dst_h.at[s], dst_v)
        pltpu.sync_copy(table_h.at[pl.ds(t0, TSTRIPE), pl.ds(co, D)],
                        table_s.at[pl.ds(t0, TSTRIPE)])
        pltpu.sync_copy(zacc_h.at[pl.ds(r0, STRIPE)], acc_s.at[pl.ds(r0, STRIPE)])
        if count:
            pltpu.sync_copy(ones_h, ones_v)
            pltpu.sync_copy(zcnt_h.at[pl.ds(r0, STRIPE)],
                            cacc_s.at[pl.ds(r0, STRIPE)])
        plsc.subcore_barrier()

        for jj in range(GAHEAD):
            pltpu.async_copy(table_s.at[src_v.at[jj]], rows_v.at[jj], gsem)

        def step(j, carry):
            p = lax.rem(j, NBUF)

            @pl.when(j >= SLAG)
            def _():
                jo = j - SLAG
                po = lax.rem(jo, NBUF)
                pltpu.make_async_copy(rows_v.at[po],
                                      acc_s.at[dst_v.at[jo]], ssem).wait()
                if count:
                    pltpu.make_async_copy(ones_v,
                                          cacc_s.at[dst_v.at[jo]],
                                          csem).wait()

            @pl.when(j + GAHEAD < STEPS)
            def _():
                jn = j + GAHEAD
                pltpu.async_copy(table_s.at[src_v.at[jn]],
                                 rows_v.at[lax.rem(jn, NBUF)], gsem)

            pltpu.make_async_copy(table_s.at[src_v.at[j]],
                                  rows_v.at[p], gsem).wait()
            pltpu.async_copy(rows_v.at[p], acc_s.at[dst_v.at[j]], ssem,
                             add=True)
            if count:
                pltpu.async_copy(ones_v, cacc_s.at[dst_v.at[j]], csem,
                                 add=True)
            return carry

        lax.fori_loop(0, STEPS, step, 0)
        for jj in range(STEPS - SLAG, STEPS):
            pltpu.make_async_copy(rows_v.at[jj % NBUF],
                                  acc_s.at[dst_v.at[jj]], ssem).wait()
            if count:
                pltpu.make_async_copy(ones_v, cacc_s.at[dst_v.at[jj]],
                                      csem).wait()
        plsc.subcore_barrier()

        pltpu.sync_copy(acc_s.at[pl.ds(r0, STRIPE)],
                        out_h.at[c, pl.ds(r0, STRIPE)])
        if count:
            @pl.when(c == 1)
            def _():
                pltpu.sync_copy(cacc_s.at[pl.ds(r0, STRIPE)],
                                cnt_h.at[pl.ds(r0, STRIPE)])

    if count:
        res = k(table, src_r, dst_r, z_acc, ones_b, z_cnt)
    else:
        res = k(table, src_r, dst_r, z_acc)
    if isinstance(res, (list, tuple)):
        return tuple(res)
    return (res,)


def _tc_mid(agg1, cnt8, xcat, bl1_2d, Wl2, Wr2, Wc):
    def body(m_ref, c_ref, xc_ref, b_ref, wl2_ref, wr2_ref, wc_ref, o_ref):
        aggsum = jnp.concatenate([m_ref[0], m_ref[1]], axis=1)
        inv = 1.0 / jnp.maximum(c_ref[:, 0:1], 1.0)
        h = jnp.maximum(aggsum * inv + b_ref[0:1, :] + xc_ref[:, H:], 0.0)
        MA = jnp.dot(wc_ref[...], wl2_ref[...],
                     preferred_element_type=jnp.float32)
        MB = jnp.dot(wc_ref[...], wr2_ref[...],
                     preferred_element_type=jnp.float32)
        dn = (((1,), (1,)), ((), ()))
        hA = lax.dot_general(h, MA, dn, preferred_element_type=jnp.float32)
        hB = lax.dot_general(h, MB, dn, preferred_element_type=jnp.float32)
        z4 = jnp.zeros((hA.shape[0], DH2 - CH), jnp.float32)
        z40 = jnp.zeros((hA.shape[0], 128 - 2 * DH2 - C), jnp.float32)
        o_ref[...] = jnp.concatenate(
            [hA[:, :CH], z4, hA[:, CH:], z4, hB, z40], axis=1)

    grid = (N // ROWS_BLK,)
    blk = lambda d: pl.BlockSpec((ROWS_BLK, d), lambda i: (i, 0))
    full = lambda a, b: pl.BlockSpec((a, b), lambda i: (0, 0))
    return pl.pallas_call(
        body,
        grid=grid,
        in_specs=[pl.BlockSpec((2, ROWS_BLK, DH1), lambda i: (0, i, 0)),
                  blk(8), blk(128), full(8, H),
                  full(OUT, H), full(OUT, H), full(C, OUT)],
        out_specs=blk(128),
        out_shape=jax.ShapeDtypeStruct((N, 128), jnp.float32),
    )(agg1, cnt8, xcat, bl1_2d, Wl2, Wr2, Wc)


def _tc_out(agg2, cnt8, hcat, bl2_2d, bc_2d, Wc):
    def body(m_ref, c_ref, hc_ref, bl_ref, bc_ref, wc_ref, o_ref):
        aggsum = jnp.concatenate([m_ref[0][:, :CH], m_ref[1][:, :CH]], axis=1)
        agg = aggsum * (1.0 / jnp.maximum(c_ref[:, 0:1], 1.0))
        dn = (((1,), (1,)), ((), ()))
        c2 = lax.dot_general(bl_ref[0:1, :], wc_ref[...], dn,
                             preferred_element_type=jnp.float32)
        o_ref[...] = (agg + hc_ref[:, 2 * DH2:2 * DH2 + C]
                      + c2 + bc_ref[0:1, :])

    grid = (N // ROWS_BLK,)
    blk = lambda d: pl.BlockSpec((ROWS_BLK, d), lambda i: (i, 0))
    full = lambda a, b: pl.BlockSpec((a, b), lambda i: (0, 0))
    return pl.pallas_call(
        body,
        grid=grid,
        in_specs=[pl.BlockSpec((2, ROWS_BLK, DH2), lambda i: (0, i, 0)),
                  blk(8), blk(128), full(8, OUT),
                  full(8, C), full(C, OUT)],
        out_specs=blk(C),
        out_shape=jax.ShapeDtypeStruct((N, C), jnp.float32),
    )(agg2, cnt8, hcat, bl2_2d, bc_2d, Wc)


def kernel(x, edge_index, Wl1, bl1, Wr1, Wl2, bl2, Wr2, Wc, bc):
    src = edge_index[0]
    dst = edge_index[1]
    pad = E_PAD - E
    src_r = jnp.concatenate([src, jnp.zeros((pad,), jnp.int32)]
                            ).reshape(NS, STEPS, BATCH)
    dst_r = jnp.concatenate([dst, jnp.full((pad,), N, jnp.int32)]
                            ).reshape(NS, STEPS, BATCH)
    z32 = jnp.zeros((N_PAD, DH1), jnp.float32)
    z24 = jnp.zeros((N_PAD, DH2), jnp.float32)
    z8 = jnp.zeros((N_PAD, 8), jnp.float32)
    ones_b = jnp.ones((BATCH, 8), jnp.float32)
    bl1_2d = jnp.broadcast_to(bl1[None, :], (8, H))
    bl2_2d = jnp.broadcast_to(bl2[None, :], (8, OUT))
    bc_2d = jnp.broadcast_to(bc[None, :], (8, C))

    xcat = _tc_in_proj(x, Wl1, Wr1)

    agg1, cnt8 = _sc_segment_mean(xcat, DH1, DH1, src_r, dst_r, z32,
                                  ones_b, z8, None)

    hcat = _tc_mid(agg1, cnt8, xcat, bl1_2d, Wl2, Wr2, Wc)

    (agg2,) = _sc_segment_mean(hcat, DH2, DH2, src_r, dst_r, z24,
                               None, None, None)

    return _tc_out(agg2, cnt8, hcat, bl2_2d, bc_2d, Wc)


# speedup vs baseline: 1.6026x; 1.0012x over previous
import functools

import jax
import jax.numpy as jnp
from jax import lax
from jax.experimental import pallas as pl
from jax.experimental.pallas import tpu as pltpu
from jax.experimental.pallas import tpu_sc as plsc

N = 10000
E = 320000
IN = 128
H = 64
OUT = 64
C = 40

NC = 2
NS = 16
BATCH = 128
STEPS = 157
EPT = STEPS * BATCH
E_PAD = EPT * NS
N_PAD = 10016
STRIPE = N_PAD // NS
TSTRIPE = N // NS
ROWS_BLK = 2000
NBUF = 8
GAHEAD = 4
SLAG = 4
DH1 = H // 2
CH = C // 2
DH2 = 24


def _tc_in_proj(x, Wl1, Wr1):
    def body(x_ref, wl_ref, wr_ref, o_ref):
        xb = x_ref[...]
        dn = (((1,), (1,)), ((), ()))
        r1 = lax.dot_general(xb, wl_ref[...], dn,
                             preferred_element_type=jnp.float32)
        r2 = lax.dot_general(xb, wr_ref[...], dn,
                             preferred_element_type=jnp.float32)
        o_ref[...] = jnp.concatenate([r1, r2], axis=1)

    grid = (N // ROWS_BLK,)
    return pl.pallas_call(
        body,
        grid=grid,
        in_specs=[
            pl.BlockSpec((ROWS_BLK, IN), lambda i: (i, 0)),
            pl.BlockSpec((H, IN), lambda i: (0, 0)),
            pl.BlockSpec((H, IN), lambda i: (0, 0)),
        ],
        out_specs=pl.BlockSpec((ROWS_BLK, 128), lambda i: (i, 0)),
        out_shape=jax.ShapeDtypeStruct((N, 128), jnp.float32),
    )(x, Wl1, Wr1)


def _sc_segment_mean(table, col_off, D, src_r, dst_r, z_acc, ones_b, z_cnt,
                     cnt_in):
    count = ones_b is not None
    mesh = plsc.VectorSubcoreMesh(core_axis_name="c", subcore_axis_name="s")

    out_type = [jax.ShapeDtypeStruct((NC, N_PAD, D), jnp.float32)]
    scratch = [
        pltpu.VMEM((STEPS, BATCH), jnp.int32),
        pltpu.VMEM((STEPS, BATCH), jnp.int32),
        pltpu.VMEM((NBUF, BATCH, D), jnp.float32),
        pltpu.VMEM_SHARED((N, D), jnp.float32),
        pltpu.VMEM_SHARED((N_PAD, D), jnp.float32),
        pltpu.SemaphoreType.DMA,
        pltpu.SemaphoreType.DMA,
    ]
    if count:
        out_type.append(jax.ShapeDtypeStruct((N_PAD, 8), jnp.float32))
        scratch.append(pltpu.VMEM((BATCH, 8), jnp.float32))
        scratch.append(pltpu.VMEM_SHARED((N_PAD, 8), jnp.float32))
        scratch.append(pltpu.SemaphoreType.DMA)

    @functools.partial(
        pl.kernel,
        mesh=mesh,
        out_type=out_type,
        scratch_types=scratch,
        compiler_params=pltpu.CompilerParams(use_tc_tiling_on_sc=False),
    )
    def k(*refs):
        if count:
            (table_h, src_h, dst_h, zacc_h, ones_h, zcnt_h,
             out_h, cnt_h, src_v, dst_v, rows_v,
             table_s, acc_s, gsem, ssem, ones_v, cacc_s, csem) = refs
        else:
            (table_h, src_h, dst_h, zacc_h,
             out_h, src_v, dst_v, rows_v,
             table_s, acc_s, gsem, ssem) = refs
        c = lax.axis_index("c")
        s = lax.axis_index("s")
        r0 = s * STRIPE
        t0 = s * TSTRIPE
        co = col_off * c

        pltpu.sync_copy(src_h.at[s], src_v)
        pltpu.sync_copy(dst_h.at[s], dst_v)
        pltpu.sync_copy(table_h.at[pl.ds(t0, TSTRIPE), pl.ds(co, D)],
                        table_s.at[pl.ds(t0, TSTRIPE)])
        pltpu.sync_copy(zacc_h.at[pl.ds(r0, STRIPE)], acc_s.at[pl.ds(r0, STRIPE)])
        if count:
            pltpu.sync_copy(ones_h, ones_v)
            pltpu.sync_copy(zcnt_h.at[pl.ds(r0, STRIPE)],
                            cacc_s.at[pl.ds(r0, STRIPE)])
        plsc.subcore_barrier()

        for jj in range(GAHEAD):
            pltpu.async_copy(table_s.at[src_v.at[jj]], rows_v.at[jj], gsem)

        def step(j, carry):
            p = lax.rem(j, NBUF)

            @pl.when(j >= SLAG)
            def _():
                jo = j - SLAG
                po = lax.rem(jo, NBUF)
                pltpu.make_async_copy(rows_v.at[po],
                                      acc_s.at[dst_v.at[jo]], ssem).wait()
                if count:
                    pltpu.make_async_copy(ones_v,
                                          cacc_s.at[dst_v.at[jo]],
                                          csem).wait()

            @pl.when(j + GAHEAD < STEPS)
            def _():
                jn = j + GAHEAD
                pltpu.async_copy(table_s.at[src_v.at[jn]],
                                 rows_v.at[lax.rem(jn, NBUF)], gsem)

            pltpu.make_async_copy(table_s.at[src_v.at[j]],
                                  rows_v.at[p], gsem).wait()
            pltpu.async_copy(rows_v.at[p], acc_s.at[dst_v.at[j]], ssem,
                             add=True)
            if count:
                pltpu.async_copy(ones_v, cacc_s.at[dst_v.at[j]], csem,
                                 add=True)
            return carry

        lax.fori_loop(0, STEPS, step, 0)
        for jj in range(STEPS - SLAG, STEPS):
            pltpu.make_async_copy(rows_v.at[jj % NBUF],
                                  acc_s.at[dst_v.at[jj]], ssem).wait()
            if count:
                pltpu.make_async_copy(ones_v, cacc_s.at[dst_v.at[jj]],
                                      csem).wait()
        plsc.subcore_barrier()

        pltpu.sync_copy(acc_s.at[pl.ds(r0, STRIPE)],
                        out_h.at[c, pl.ds(r0, STRIPE)])
        if count:
            @pl.when(c == 1)
            def _():
                pltpu.sync_copy(cacc_s.at[pl.ds(r0, STRIPE)],
                                cnt_h.at[pl.ds(r0, STRIPE)])

    if count:
        res = k(table, src_r, dst_r, z_acc, ones_b, z_cnt)
    else:
        res = k(table, src_r, dst_r, z_acc)
    if isinstance(res, (list, tuple)):
        return tuple(res)
    return (res,)


def _tc_mid(agg1, cnt8, xcat, bl1_2d, Wl2, Wr2, Wc):
    def body(m_ref, c_ref, xc_ref, b_ref, wl2_ref, wr2_ref, wc_ref, o_ref):
        aggsum = jnp.concatenate([m_ref[0], m_ref[1]], axis=1)
        inv = 1.0 / jnp.maximum(c_ref[:, 0:1], 1.0)
        h = jnp.maximum(aggsum * inv + b_ref[0:1, :] + xc_ref[:, H:], 0.0)
        MA = jnp.dot(wc_ref[...], wl2_ref[...],
                     preferred_element_type=jnp.float32)
        MB = jnp.dot(wc_ref[...], wr2_ref[...],
                     preferred_element_type=jnp.float32)
        dn = (((1,), (1,)), ((), ()))
        hA = lax.dot_general(h, MA, dn, preferred_element_type=jnp.float32)
        hB = lax.dot_general(h, MB, dn, preferred_element_type=jnp.float32)
        z4 = jnp.zeros((hA.shape[0], DH2 - CH), jnp.float32)
        z40 = jnp.zeros((hA.shape[0], 128 - 2 * DH2 - C), jnp.float32)
        o_ref[...] = jnp.concatenate(
            [hA[:, :CH], z4, hA[:, CH:], z4, hB, z40], axis=1)

    grid = (N // ROWS_BLK,)
    blk = lambda d: pl.BlockSpec((ROWS_BLK, d), lambda i: (i, 0))
    full = lambda a, b: pl.BlockSpec((a, b), lambda i: (0, 0))
    return pl.pallas_call(
        body,
        grid=grid,
        in_specs=[pl.BlockSpec((2, ROWS_BLK, DH1), lambda i: (0, i, 0)),
                  blk(8), blk(128), full(8, H),
                  full(OUT, H), full(OUT, H), full(C, OUT)],
        out_specs=blk(128),
        out_shape=jax.ShapeDtypeStruct((N, 128), jnp.float32),
    )(agg1, cnt8, xcat, bl1_2d, Wl2, Wr2, Wc)


def _tc_out(agg2, cnt8, hcat, bl2_2d, bc_2d, Wc):
    def body(m_ref, c_ref, hc_ref, bl_ref, bc_ref, wc_ref, o_ref):
        aggsum = jnp.concatenate([m_ref[0][:, :CH], m_ref[1][:, :CH]], axis=1)
        agg = aggsum * (1.0 / jnp.maximum(c_ref[:, 0:1], 1.0))
        dn = (((1,), (1,)), ((), ()))
        c2 = lax.dot_general(bl_ref[0:1, :], wc_ref[...], dn,
                             preferred_element_type=jnp.float32)
        o_ref[...] = (agg + hc_ref[:, 2 * DH2:2 * DH2 + C]
                      + c2 + bc_ref[0:1, :])

    grid = (N // ROWS_BLK,)
    blk = lambda d: pl.BlockSpec((ROWS_BLK, d), lambda i: (i, 0))
    full = lambda a, b: pl.BlockSpec((a, b), lambda i: (0, 0))
    return pl.pallas_call(
        body,
        grid=grid,
        in_specs=[pl.BlockSpec((2, ROWS_BLK, DH2), lambda i: (0, i, 0)),
                  blk(8), blk(128), full(8, OUT),
                  full(8, C), full(C, OUT)],
        out_specs=blk(C),
        out_shape=jax.ShapeDtypeStruct((N, C), jnp.float32),
    )(agg2, cnt8, hcat, bl2_2d, bc_2d, Wc)


def kernel(x, edge_index, Wl1, bl1, Wr1, Wl2, bl2, Wr2, Wc, bc):
    src = edge_index[0]
    dst = edge_index[1]
    pad = E_PAD - E
    src_r = jnp.concatenate([src, jnp.zeros((pad,), jnp.int32)]
                            ).reshape(NS, STEPS, BATCH)
    dst_r = jnp.concatenate([dst, jnp.full((pad,), N, jnp.int32)]
                            ).reshape(NS, STEPS, BATCH)
    z32 = jnp.zeros((N_PAD, DH1), jnp.float32)
    z24 = jnp.zeros((N_PAD, DH2), jnp.float32)
    z8 = jnp.zeros((N_PAD, 8), jnp.float32)
    ones_b = jnp.ones((BATCH, 8), jnp.float32)
    bl1_2d = jnp.broadcast_to(bl1[None, :], (8, H))
    bl2_2d = jnp.broadcast_to(bl2[None, :], (8, OUT))
    bc_2d = jnp.broadcast_to(bc[None, :], (8, C))

    xcat = _tc_in_proj(x, Wl1, Wr1)

    agg1, cnt8 = _sc_segment_mean(xcat, DH1, DH1, src_r, dst_r, z32,
                                  ones_b, z8, None)

    hcat = _tc_mid(agg1, cnt8, xcat, bl1_2d, Wl2, Wr2, Wc)

    (agg2,) = _sc_segment_mean(hcat, DH2, DH2, src_r, dst_r, z24,
                               None, None, None)

    return _tc_out(agg2, cnt8, hcat, bl2_2d, bc_2d, Wc)


# speedup vs baseline: 1.6228x; 1.0126x over previous
import functools

import jax
import jax.numpy as jnp
from jax import lax
from jax.experimental import pallas as pl
from jax.experimental.pallas import tpu as pltpu
from jax.experimental.pallas import tpu_sc as plsc

N = 10000
E = 320000
IN = 128
H = 64
OUT = 64
C = 40

NC = 2
NS = 16
BATCH = 128
STEPS = 157
EPT = STEPS * BATCH
E_PAD = EPT * NS
N_PAD = 10016
STRIPE = N_PAD // NS
TSTRIPE = N // NS
ROWS_BLK = 2000
NBUF = 8
GAHEAD = 4
SLAG = 4
DH1 = H // 2
CH = C // 2
DH2 = 24


def _tc_in_proj(x, Wl1, Wr1):
    def body(x_ref, wl_ref, wr_ref, o_ref):
        xb = x_ref[...]
        dn = (((1,), (1,)), ((), ()))
        r1 = lax.dot_general(xb, wl_ref[...], dn,
                             preferred_element_type=jnp.float32)
        r2 = lax.dot_general(xb, wr_ref[...], dn,
                             preferred_element_type=jnp.float32)
        o_ref[...] = jnp.concatenate([r1, r2], axis=1)

    grid = (N // ROWS_BLK,)
    return pl.pallas_call(
        body,
        grid=grid,
        in_specs=[
            pl.BlockSpec((ROWS_BLK, IN), lambda i: (i, 0)),
            pl.BlockSpec((H, IN), lambda i: (0, 0)),
            pl.BlockSpec((H, IN), lambda i: (0, 0)),
        ],
        out_specs=pl.BlockSpec((ROWS_BLK, 128), lambda i: (i, 0)),
        out_shape=jax.ShapeDtypeStruct((N, 128), jnp.float32),
    )(x, Wl1, Wr1)


def _sc_segment_mean(table, col_off, D, src_r, dst_r, z_acc, ones_b, z_cnt,
                     cnt_in):
    count = ones_b is not None
    mesh = plsc.VectorSubcoreMesh(core_axis_name="c", subcore_axis_name="s")

    out_type = [jax.ShapeDtypeStruct((NC, N_PAD, D), jnp.float32)]
    scratch = [
        pltpu.VMEM((STEPS, BATCH), jnp.int32),
        pltpu.VMEM((STEPS, BATCH), jnp.int32),
        pltpu.VMEM((NBUF, BATCH, D), jnp.float32),
        pltpu.VMEM_SHARED((N, D), jnp.float32),
        pltpu.VMEM_SHARED((N_PAD, D), jnp.float32),
        pltpu.SemaphoreType.DMA,
        pltpu.SemaphoreType.DMA,
    ]
    if count:
        out_type.append(jax.ShapeDtypeStruct((NC, N_PAD, 8), jnp.float32))
        scratch.append(pltpu.VMEM((BATCH, 8), jnp.float32))
        scratch.append(pltpu.VMEM_SHARED((N_PAD, 8), jnp.float32))
        scratch.append(pltpu.SemaphoreType.DMA)

    @functools.partial(
        pl.kernel,
        mesh=mesh,
        out_type=out_type,
        scratch_types=scratch,
        compiler_params=pltpu.CompilerParams(use_tc_tiling_on_sc=False),
    )
    def k(*refs):
        if count:
            (table_h, src_h, dst_h, zacc_h, ones_h, zcnt_h,
             out_h, cnt_h, src_v, dst_v, rows_v,
             table_s, acc_s, gsem, ssem, ones_v, cacc_s, csem) = refs
        else:
            (table_h, src_h, dst_h, zacc_h,
             out_h, src_v, dst_v, rows_v,
             table_s, acc_s, gsem, ssem) = refs
        c = lax.axis_index("c")
        s = lax.axis_index("s")
        r0 = s * STRIPE
        t0 = s * TSTRIPE
        co = col_off * c

        pltpu.sync_copy(src_h.at[s], src_v)
        pltpu.sync_copy(dst_h.at[s], dst_v)
        pltpu.sync_copy(table_h.at[pl.ds(t0, TSTRIPE), pl.ds(co, D)],
                        table_s.at[pl.ds(t0, TSTRIPE)])
        pltpu.sync_copy(zacc_h.at[pl.ds(r0, STRIPE)], acc_s.at[pl.ds(r0, STRIPE)])
        if count:
            pltpu.sync_copy(ones_h, ones_v)
            pltpu.sync_copy(zcnt_h.at[pl.ds(r0, STRIPE)],
                            cacc_s.at[pl.ds(r0, STRIPE)])
        plsc.subcore_barrier()

        for jj in range(GAHEAD):
            pltpu.async_copy(table_s.at[src_v.at[jj]], rows_v.at[jj], gsem)

        def step(j, carry):
            p = lax.rem(j, NBUF)

            @pl.when(j >= SLAG)
            def _():
                jo = j - SLAG
                po = lax.rem(jo, NBUF)
                pltpu.make_async_copy(rows_v.at[po],
                                      acc_s.at[dst_v.at[jo]], ssem).wait()
                if count:
                    @pl.when(lax.rem(jo, 2) == c)
                    def _():
                        pltpu.make_async_copy(ones_v,
                                              cacc_s.at[dst_v.at[jo]],
                                              csem).wait()

            @pl.when(j + GAHEAD < STEPS)
            def _():
                jn = j + GAHEAD
                pltpu.async_copy(table_s.at[src_v.at[jn]],
                                 rows_v.at[lax.rem(jn, NBUF)], gsem)

            pltpu.make_async_copy(table_s.at[src_v.at[j]],
                                  rows_v.at[p], gsem).wait()
            pltpu.async_copy(rows_v.at[p], acc_s.at[dst_v.at[j]], ssem,
                             add=True)
            if count:
                @pl.when(lax.rem(j, 2) == c)
                def _():
                    pltpu.async_copy(ones_v, cacc_s.at[dst_v.at[j]], csem,
                                     add=True)
            return carry

        lax.fori_loop(0, STEPS, step, 0)
        for jj in range(STEPS - SLAG, STEPS):
            pltpu.make_async_copy(rows_v.at[jj % NBUF],
                                  acc_s.at[dst_v.at[jj]], ssem).wait()
            if count:
                @pl.when(lax.rem(jj, 2) == c)
                def _():
                    pltpu.make_async_copy(ones_v, cacc_s.at[dst_v.at[jj]],
                                          csem).wait()
        plsc.subcore_barrier()

        pltpu.sync_copy(acc_s.at[pl.ds(r0, STRIPE)],
                        out_h.at[c, pl.ds(r0, STRIPE)])
        if count:
            pltpu.sync_copy(cacc_s.at[pl.ds(r0, STRIPE)],
                            cnt_h.at[c, pl.ds(r0, STRIPE)])

    if count:
        res = k(table, src_r, dst_r, z_acc, ones_b, z_cnt)
    else:
        res = k(table, src_r, dst_r, z_acc)
    if isinstance(res, (list, tuple)):
        return tuple(res)
    return (res,)


def _tc_mid(agg1, cnt8, xcat, bl1_2d, Wl2, Wr2, Wc):
    def body(m_ref, c_ref, xc_ref, b_ref, wl2_ref, wr2_ref, wc_ref, o_ref):
        aggsum = jnp.concatenate([m_ref[0], m_ref[1]], axis=1)
        cnt = c_ref[0, :, 0:1] + c_ref[1, :, 0:1]
        inv = 1.0 / jnp.maximum(cnt, 1.0)
        h = jnp.maximum(aggsum * inv + b_ref[0:1, :] + xc_ref[:, H:], 0.0)
        MA = jnp.dot(wc_ref[...], wl2_ref[...],
                     preferred_element_type=jnp.float32)
        MB = jnp.dot(wc_ref[...], wr2_ref[...],
                     preferred_element_type=jnp.float32)
        dn = (((1,), (1,)), ((), ()))
        hA = lax.dot_general(h, MA, dn, preferred_element_type=jnp.float32)
        hB = lax.dot_general(h, MB, dn, preferred_element_type=jnp.float32)
        z4 = jnp.zeros((hA.shape[0], DH2 - CH), jnp.float32)
        z40 = jnp.zeros((hA.shape[0], 128 - 2 * DH2 - C), jnp.float32)
        o_ref[...] = jnp.concatenate(
            [hA[:, :CH], z4, hA[:, CH:], z4, hB, z40], axis=1)

    grid = (N // ROWS_BLK,)
    blk = lambda d: pl.BlockSpec((ROWS_BLK, d), lambda i: (i, 0))
    full = lambda a, b: pl.BlockSpec((a, b), lambda i: (0, 0))
    return pl.pallas_call(
        body,
        grid=grid,
        in_specs=[pl.BlockSpec((2, ROWS_BLK, DH1), lambda i: (0, i, 0)),
                  pl.BlockSpec((2, ROWS_BLK, 8), lambda i: (0, i, 0)),
                  blk(128), full(8, H),
                  full(OUT, H), full(OUT, H), full(C, OUT)],
        out_specs=blk(128),
        out_shape=jax.ShapeDtypeStruct((N, 128), jnp.float32),
    )(agg1, cnt8, xcat, bl1_2d, Wl2, Wr2, Wc)


def _tc_out(agg2, cnt8, hcat, bl2_2d, bc_2d, Wc):
    def body(m_ref, c_ref, hc_ref, bl_ref, bc_ref, wc_ref, o_ref):
        aggsum = jnp.concatenate([m_ref[0][:, :CH], m_ref[1][:, :CH]], axis=1)
        cnt = c_ref[0, :, 0:1] + c_ref[1, :, 0:1]
        agg = aggsum * (1.0 / jnp.maximum(cnt, 1.0))
        dn = (((1,), (1,)), ((), ()))
        c2 = lax.dot_general(bl_ref[0:1, :], wc_ref[...], dn,
                             preferred_element_type=jnp.float32)
        o_ref[...] = (agg + hc_ref[:, 2 * DH2:2 * DH2 + C]
                      + c2 + bc_ref[0:1, :])

    grid = (N // ROWS_BLK,)
    blk = lambda d: pl.BlockSpec((ROWS_BLK, d), lambda i: (i, 0))
    full = lambda a, b: pl.BlockSpec((a, b), lambda i: (0, 0))
    return pl.pallas_call(
        body,
        grid=grid,
        in_specs=[pl.BlockSpec((2, ROWS_BLK, DH2), lambda i: (0, i, 0)),
                  pl.BlockSpec((2, ROWS_BLK, 8), lambda i: (0, i, 0)),
                  blk(128), full(8, OUT),
                  full(8, C), full(C, OUT)],
        out_specs=blk(C),
        out_shape=jax.ShapeDtypeStruct((N, C), jnp.float32),
    )(agg2, cnt8, hcat, bl2_2d, bc_2d, Wc)


def kernel(x, edge_index, Wl1, bl1, Wr1, Wl2, bl2, Wr2, Wc, bc):
    src = edge_index[0]
    dst = edge_index[1]
    pad = E_PAD - E
    src_r = jnp.concatenate([src, jnp.zeros((pad,), jnp.int32)]
                            ).reshape(NS, STEPS, BATCH)
    dst_r = jnp.concatenate([dst, jnp.full((pad,), N, jnp.int32)]
                            ).reshape(NS, STEPS, BATCH)
    z32 = jnp.zeros((N_PAD, DH1), jnp.float32)
    z24 = jnp.zeros((N_PAD, DH2), jnp.float32)
    z8 = jnp.zeros((N_PAD, 8), jnp.float32)
    ones_b = jnp.ones((BATCH, 8), jnp.float32)
    bl1_2d = jnp.broadcast_to(bl1[None, :], (8, H))
    bl2_2d = jnp.broadcast_to(bl2[None, :], (8, OUT))
    bc_2d = jnp.broadcast_to(bc[None, :], (8, C))

    xcat = _tc_in_proj(x, Wl1, Wr1)

    agg1, cnt8 = _sc_segment_mean(xcat, DH1, DH1, src_r, dst_r, z32,
                                  ones_b, z8, None)

    hcat = _tc_mid(agg1, cnt8, xcat, bl1_2d, Wl2, Wr2, Wc)

    (agg2,) = _sc_segment_mean(hcat, DH2, DH2, src_r, dst_r, z24,
                               None, None, None)

    return _tc_out(agg2, cnt8, hcat, bl2_2d, bc_2d, Wc)


# speedup vs baseline: 1.7055x; 1.0510x over previous
import functools

import jax
import jax.numpy as jnp
from jax import lax
from jax.experimental import pallas as pl
from jax.experimental.pallas import tpu as pltpu
from jax.experimental.pallas import tpu_sc as plsc

N = 10000
E = 320000
IN = 128
H = 64
OUT = 64
C = 40

NC = 2
NS = 16
BATCH = 128
STEPS = 157
EPT = STEPS * BATCH
E_PAD = EPT * NS
N_PAD = 10016
STRIPE = N_PAD // NS
TSTRIPE = N // NS
ROWS_BLK = 2000
NBUF = 8
GAHEAD = 4
SLAG = 4
DH1 = H // 2
CH = C // 2
DH2 = 24


def _tc_in_proj(x, Wl1, Wr1):
    def body(x_ref, wl_ref, wr_ref, o_ref):
        xb = x_ref[...]
        dn = (((1,), (1,)), ((), ()))
        r1 = lax.dot_general(xb, wl_ref[...], dn,
                             preferred_element_type=jnp.float32)
        r2 = lax.dot_general(xb, wr_ref[...], dn,
                             preferred_element_type=jnp.float32)
        o_ref[...] = jnp.concatenate([r1, r2], axis=1)

    grid = (N // ROWS_BLK,)
    return pl.pallas_call(
        body,
        grid=grid,
        in_specs=[
            pl.BlockSpec((ROWS_BLK, IN), lambda i: (i, 0)),
            pl.BlockSpec((H, IN), lambda i: (0, 0)),
            pl.BlockSpec((H, IN), lambda i: (0, 0)),
        ],
        out_specs=pl.BlockSpec((ROWS_BLK, 128), lambda i: (i, 0)),
        out_shape=jax.ShapeDtypeStruct((N, 128), jnp.float32),
    )(x, Wl1, Wr1)


def _sc_segment_mean(table, col_off, D, src_r, dst_r, z_acc, ones_b, z_cnt,
                     cnt_in):
    count = ones_b is not None
    mesh = plsc.VectorSubcoreMesh(core_axis_name="c", subcore_axis_name="s")

    out_type = [jax.ShapeDtypeStruct((NC, N_PAD, D), jnp.float32)]
    scratch = [
        pltpu.VMEM((STEPS, BATCH), jnp.int32),
        pltpu.VMEM((STEPS, BATCH), jnp.int32),
        pltpu.VMEM((NBUF, BATCH, D), jnp.float32),
        pltpu.VMEM_SHARED((N_PAD, D), jnp.float32),
        pltpu.VMEM_SHARED((N_PAD, D), jnp.float32),
        pltpu.SemaphoreType.DMA,
        pltpu.SemaphoreType.DMA,
    ]
    if count:
        out_type.append(jax.ShapeDtypeStruct((NC, N_PAD, 8), jnp.float32))
        scratch.append(pltpu.VMEM((BATCH, 8), jnp.float32))
        scratch.append(pltpu.VMEM_SHARED((N_PAD, 8), jnp.float32))
        scratch.append(pltpu.SemaphoreType.DMA)

    @functools.partial(
        pl.kernel,
        mesh=mesh,
        out_type=out_type,
        scratch_types=scratch,
        compiler_params=pltpu.CompilerParams(use_tc_tiling_on_sc=False),
    )
    def k(*refs):
        if count:
            (table_h, edges_h, zacc_h, ones_h, zcnt_h,
             out_h, cnt_h, src_v, dst_v, rows_v,
             table_s, acc_s, gsem, ssem, ones_v, cacc_s, csem) = refs
        else:
            (table_h, edges_h, zacc_h,
             out_h, src_v, dst_v, rows_v,
             table_s, acc_s, gsem, ssem) = refs
        c = lax.axis_index("c")
        s = lax.axis_index("s")
        r0 = s * STRIPE
        t0 = s * TSTRIPE
        co = col_off * c

        pltpu.sync_copy(edges_h.at[0, s], src_v)
        pltpu.sync_copy(edges_h.at[1, s], dst_v)
        pltpu.sync_copy(table_h.at[pl.ds(t0, TSTRIPE), pl.ds(co, D)],
                        table_s.at[pl.ds(t0, TSTRIPE)])
        pltpu.sync_copy(zacc_h.at[pl.ds(r0, STRIPE)], acc_s.at[pl.ds(r0, STRIPE)])
        if count:
            pltpu.sync_copy(ones_h, ones_v)
            pltpu.sync_copy(zcnt_h.at[pl.ds(r0, STRIPE)],
                            cacc_s.at[pl.ds(r0, STRIPE)])
        plsc.subcore_barrier()

        for jj in range(GAHEAD):
            pltpu.async_copy(table_s.at[src_v.at[jj]], rows_v.at[jj], gsem)

        def step(j, carry):
            p = lax.rem(j, NBUF)

            @pl.when(j >= SLAG)
            def _():
                jo = j - SLAG
                po = lax.rem(jo, NBUF)
                pltpu.make_async_copy(rows_v.at[po],
                                      acc_s.at[dst_v.at[jo]], ssem).wait()
                if count:
                    @pl.when(lax.rem(jo, 2) == c)
                    def _():
                        pltpu.make_async_copy(ones_v,
                                              cacc_s.at[dst_v.at[jo]],
                                              csem).wait()

            @pl.when(j + GAHEAD < STEPS)
            def _():
                jn = j + GAHEAD
                pltpu.async_copy(table_s.at[src_v.at[jn]],
                                 rows_v.at[lax.rem(jn, NBUF)], gsem)

            pltpu.make_async_copy(table_s.at[src_v.at[j]],
                                  rows_v.at[p], gsem).wait()
            pltpu.async_copy(rows_v.at[p], acc_s.at[dst_v.at[j]], ssem,
                             add=True)
            if count:
                @pl.when(lax.rem(j, 2) == c)
                def _():
                    pltpu.async_copy(ones_v, cacc_s.at[dst_v.at[j]], csem,
                                     add=True)
            return carry

        lax.fori_loop(0, STEPS, step, 0)
        for jj in range(STEPS - SLAG, STEPS):
            pltpu.make_async_copy(rows_v.at[jj % NBUF],
                                  acc_s.at[dst_v.at[jj]], ssem).wait()
            if count:
                @pl.when(lax.rem(jj, 2) == c)
                def _():
                    pltpu.make_async_copy(ones_v, cacc_s.at[dst_v.at[jj]],
                                          csem).wait()
        plsc.subcore_barrier()

        pltpu.sync_copy(acc_s.at[pl.ds(r0, STRIPE)],
                        out_h.at[c, pl.ds(r0, STRIPE)])
        if count:
            pltpu.sync_copy(cacc_s.at[pl.ds(r0, STRIPE)],
                            cnt_h.at[c, pl.ds(r0, STRIPE)])

    if count:
        res = k(table, src_r, z_acc, ones_b, z_cnt)
    else:
        res = k(table, src_r, z_acc)
    if isinstance(res, (list, tuple)):
        return tuple(res)
    return (res,)


def _tc_mid(agg1, cnt8, xcat, bl1_2d, Wl2, Wr2, Wc):
    def body(m_ref, c_ref, xc_ref, b_ref, wl2_ref, wr2_ref, wc_ref, o_ref):
        aggsum = jnp.concatenate([m_ref[0], m_ref[1]], axis=1)
        cnt = c_ref[0, :, 0:1] + c_ref[1, :, 0:1]
        inv = 1.0 / jnp.maximum(cnt, 1.0)
        h = jnp.maximum(aggsum * inv + b_ref[0:1, :] + xc_ref[:, H:], 0.0)
        MA = jnp.dot(wc_ref[...], wl2_ref[...],
                     preferred_element_type=jnp.float32)
        MB = jnp.dot(wc_ref[...], wr2_ref[...],
                     preferred_element_type=jnp.float32)
        dn = (((1,), (1,)), ((), ()))
        hA = lax.dot_general(h, MA, dn, preferred_element_type=jnp.float32)
        hB = lax.dot_general(h, MB, dn, preferred_element_type=jnp.float32)
        z4 = jnp.zeros((hA.shape[0], DH2 - CH), jnp.float32)
        z40 = jnp.zeros((hA.shape[0], 128 - 2 * DH2 - C), jnp.float32)
        o_ref[...] = jnp.concatenate(
            [hA[:, :CH], z4, hA[:, CH:], z4, hB, z40], axis=1)

    grid = (N // ROWS_BLK,)
    blk = lambda d: pl.BlockSpec((ROWS_BLK, d), lambda i: (i, 0))
    full = lambda a, b: pl.BlockSpec((a, b), lambda i: (0, 0))
    return pl.pallas_call(
        body,
        grid=grid,
        in_specs=[pl.BlockSpec((2, ROWS_BLK, DH1), lambda i: (0, i, 0)),
                  pl.BlockSpec((2, ROWS_BLK, 8), lambda i: (0, i, 0)),
                  blk(128), full(8, H),
                  full(OUT, H), full(OUT, H), full(C, OUT)],
        out_specs=blk(128),
        out_shape=jax.ShapeDtypeStruct((N, 128), jnp.float32),
    )(agg1, cnt8, xcat, bl1_2d, Wl2, Wr2, Wc)


def _tc_out(agg2, cnt8, hcat, bl2_2d, bc_2d, Wc):
    def body(m_ref, c_ref, hc_ref, bl_ref, bc_ref, wc_ref, o_ref):
        aggsum = jnp.concatenate([m_ref[0][:, :CH], m_ref[1][:, :CH]], axis=1)
        cnt = c_ref[0, :, 0:1] + c_ref[1, :, 0:1]
        agg = aggsum * (1.0 / jnp.maximum(cnt, 1.0))
        dn = (((1,), (1,)), ((), ()))
        c2 = lax.dot_general(bl_ref[0:1, :], wc_ref[...], dn,
                             preferred_element_type=jnp.float32)
        o_ref[...] = (agg + hc_ref[:, 2 * DH2:2 * DH2 + C]
                      + c2 + bc_ref[0:1, :])

    grid = (N // ROWS_BLK,)
    blk = lambda d: pl.BlockSpec((ROWS_BLK, d), lambda i: (i, 0))
    full = lambda a, b: pl.BlockSpec((a, b), lambda i: (0, 0))
    return pl.pallas_call(
        body,
        grid=grid,
        in_specs=[pl.BlockSpec((2, ROWS_BLK, DH2), lambda i: (0, i, 0)),
                  pl.BlockSpec((2, ROWS_BLK, 8), lambda i: (0, i, 0)),
                  blk(128), full(8, OUT),
                  full(8, C), full(C, OUT)],
        out_specs=blk(C),
        out_shape=jax.ShapeDtypeStruct((N, C), jnp.float32),
    )(agg2, cnt8, hcat, bl2_2d, bc_2d, Wc)


def kernel(x, edge_index, Wl1, bl1, Wr1, Wl2, bl2, Wr2, Wc, bc):
    pad = E_PAD - E
    src_r = jnp.pad(edge_index, ((0, 0), (0, pad)), constant_values=N
                    ).reshape(2, NS, STEPS, BATCH)
    z32 = jnp.zeros((N_PAD, DH1), jnp.float32)
    z24 = jnp.zeros((N_PAD, DH2), jnp.float32)
    z8 = jnp.zeros((N_PAD, 8), jnp.float32)
    ones_b = jnp.ones((BATCH, 8), jnp.float32)
    bl1_2d = jnp.broadcast_to(bl1[None, :], (8, H))
    bl2_2d = jnp.broadcast_to(bl2[None, :], (8, OUT))
    bc_2d = jnp.broadcast_to(bc[None, :], (8, C))

    xcat = _tc_in_proj(x, Wl1, Wr1)

    agg1, cnt8 = _sc_segment_mean(xcat, DH1, DH1, src_r, None, z32,
                                  ones_b, z8, None)

    hcat = _tc_mid(agg1, cnt8, xcat, bl1_2d, Wl2, Wr2, Wc)

    (agg2,) = _sc_segment_mean(hcat, DH2, DH2, src_r, None, z24,
                               None, None, None)

    return _tc_out(agg2, cnt8, hcat, bl2_2d, bc_2d, Wc)
